# Initial kernel scaffold; baseline (speedup 1.0000x reference)
import jax
import jax.numpy as jnp
from jax.experimental import pallas as pl


def kernel(x_enc, x_mark_enc, x_dec, x_mark_dec, params):
    raise NotImplementedError("write your pallas kernel here")


# speedup vs baseline: 8.0847x; 8.0847x over previous
import math

import jax
import jax.numpy as jnp
import numpy as np
from jax.experimental import pallas as pl
from jax.experimental.pallas import tpu as pltpu

_B = 8
_SEQ = 512
_LABEL = 96
_PRED = 192
_DEC = _LABEL + _PRED
_CIN = 321
_DM = 512
_DFF = 2048
_MA = 25
_PAD = (_MA - 1) // 2
_FACTOR = 3

_PREC = jax.lax.Precision.HIGHEST


def _pos_encoding(L, d):
    pe = np.zeros((L, d), dtype=np.float32)
    pos = np.arange(L, dtype=np.float32)[:, None]
    div = np.exp(np.arange(0, d, 2, dtype=np.float32) * (-math.log(10000.0) / d))
    pe[:, 0::2] = np.sin(pos * div)
    pe[:, 1::2] = np.cos(pos * div)
    return jnp.asarray(pe)


def _dft_mats(L):
    t = np.arange(L)
    ft = np.outer(t, t) % L
    ang = (2.0 * np.pi / L) * ft.astype(np.float64)
    return (jnp.asarray(np.cos(ang), dtype=jnp.float32),
            jnp.asarray(np.sin(ang), dtype=jnp.float32))


def _batch_spec(shape):
    nd = len(shape)
    return pl.BlockSpec((1,) + tuple(shape[1:]),
                        lambda b, _nd=nd: (b,) + (0,) * (_nd - 1))


def _full_spec(shape):
    nd = len(shape)
    return pl.BlockSpec(tuple(shape), lambda b, _nd=nd: (0,) * _nd)


def _moving_avg_2d(x):
    L, C = x.shape
    front = jnp.broadcast_to(x[0:1], (_PAD, C))
    end = jnp.broadcast_to(x[L - 1:L], (_PAD, C))
    xp = jnp.concatenate([front, x, end], axis=0)
    acc = xp[0:L]
    for j in range(1, _MA):
        acc = acc + xp[j:j + L]
    return acc * (1.0 / _MA)


def _head_body(x_ref, seas_ref, ma_ref, mean_ref):
    x = x_ref[0]
    ma = _moving_avg_2d(x)
    seas_ref[0] = x - ma
    ma_ref[0] = ma
    mean_ref[0] = jnp.mean(x, axis=0, keepdims=True)


def _head_decomp(x):
    Bq, L, C = x.shape
    return pl.pallas_call(
        _head_body,
        grid=(Bq,),
        in_specs=[_batch_spec(x.shape)],
        out_specs=[_batch_spec((Bq, L, C)), _batch_spec((Bq, L, C)),
                   _batch_spec((Bq, 1, C))],
        out_shape=[jax.ShapeDtypeStruct((Bq, L, C), jnp.float32),
                   jax.ShapeDtypeStruct((Bq, L, C), jnp.float32),
                   jax.ShapeDtypeStruct((Bq, 1, C), jnp.float32)],
    )(x)


def _embed_body(x_ref, mark_ref, w3_ref, te_ref, pe_ref, out_ref):
    x = x_ref[0]
    xm1 = jnp.concatenate([x[-1:], x[:-1]], axis=0)
    xp1 = jnp.concatenate([x[1:], x[:1]], axis=0)
    acc = jnp.dot(xm1, w3_ref[0], precision=_PREC)
    acc = acc + jnp.dot(x, w3_ref[1], precision=_PREC)
    acc = acc + jnp.dot(xp1, w3_ref[2], precision=_PREC)
    acc = acc + jnp.dot(mark_ref[0], te_ref[...], precision=_PREC)
    out_ref[0] = acc + pe_ref[...]


def _embed(x, mark, w3, te, pe):
    Bq, L, _ = x.shape
    return pl.pallas_call(
        _embed_body,
        grid=(Bq,),
        in_specs=[_batch_spec(x.shape), _batch_spec(mark.shape),
                  _full_spec(w3.shape), _full_spec(te.shape),
                  _full_spec(pe.shape)],
        out_specs=_batch_spec((Bq, L, _DM)),
        out_shape=jax.ShapeDtypeStruct((Bq, L, _DM), jnp.float32),
    )(x, mark, w3, te, pe)


def _make_attn_body(L, topk, scale):
    def body(xq_ref, xkv_ref, wq_ref, bq_ref, wk_ref, bk_ref, wv_ref, bv_ref,
             wo_ref, bo_ref, cf_ref, sf_ref, out_ref):
        xq = xq_ref[0]
        xkv = xkv_ref[0]
        q = jnp.dot(xq, wq_ref[...], precision=_PREC) + bq_ref[...]
        k = jnp.dot(xkv, wk_ref[...], precision=_PREC) + bk_ref[...]
        v = jnp.dot(xkv, wv_ref[...], precision=_PREC) + bv_ref[...]

        cf = cf_ref[...]
        sf = sf_ref[...]
        qr = jnp.dot(cf, q, precision=_PREC)
        qi = -jnp.dot(sf, q, precision=_PREC)
        kr = jnp.dot(cf, k, precision=_PREC)
        ki = -jnp.dot(sf, k, precision=_PREC)
        pr = jnp.sum(qr * kr + qi * ki, axis=1, keepdims=True)
        pi = jnp.sum(qi * kr - qr * ki, axis=1, keepdims=True)
        mv = (jnp.dot(cf, pr, precision=_PREC)
              - jnp.dot(sf, pi, precision=_PREC)) * scale

        iota = jax.lax.broadcasted_iota(jnp.int32, (L, 1), 0)
        work = mv
        sel = jnp.zeros((L, 1), jnp.bool_)
        m0 = None
        for i in range(topk):
            m = jnp.max(work)
            if i == 0:
                m0 = m
            idx = jnp.min(jnp.where(work == m, iota, L))
            onehot = iota == idx
            sel = jnp.logical_or(sel, onehot)
            work = jnp.where(onehot, -jnp.inf, work)

        e = jnp.where(sel, jnp.exp(mv - m0), 0.0)
        cvec = e / jnp.sum(e)

        cr = jnp.dot(cf, cvec, precision=_PREC)
        ci = -jnp.dot(sf, cvec, precision=_PREC)
        vr = jnp.dot(cf, v, precision=_PREC)
        vi = -jnp.dot(sf, v, precision=_PREC)
        re = cr * vr + ci * vi
        im = cr * vi - ci * vr
        agg = (jnp.dot(cf, re, precision=_PREC)
               - jnp.dot(sf, im, precision=_PREC)) * (1.0 / L)

        out_ref[0] = jnp.dot(agg, wo_ref[...], precision=_PREC) + bo_ref[...]

    return body


def _attn(xq, xkv, ap, L):
    topk = int(_FACTOR * math.log(L))
    scale = 1.0 / (L * _DM)
    cf, sf = _dft_mats(L)
    b2 = lambda v: v.reshape(1, -1)
    args = (xq, xkv, ap['Wq'], b2(ap['bq']), ap['Wk'], b2(ap['bk']),
            ap['Wv'], b2(ap['bv']), ap['Wo'], b2(ap['bo']), cf, sf)
    in_specs = [_batch_spec(xq.shape), _batch_spec(xkv.shape)] + \
               [_full_spec(a.shape) for a in args[2:]]
    return pl.pallas_call(
        _make_attn_body(L, topk, scale),
        grid=(xq.shape[0],),
        in_specs=in_specs,
        out_specs=_batch_spec((xq.shape[0], L, _DM)),
        out_shape=jax.ShapeDtypeStruct((xq.shape[0], L, _DM), jnp.float32),
    )(*args)


def _enc_ffn_body(x_ref, a_ref, w1_ref, b1_ref, w2_ref, b2_ref, out_ref):
    u = x_ref[0] + a_ref[0]
    x1 = u - _moving_avg_2d(u)
    h = jax.nn.gelu(jnp.dot(x1, w1_ref[...], precision=_PREC) + b1_ref[...])
    y = jnp.dot(h, w2_ref[...], precision=_PREC) + b2_ref[...]
    s = x1 + y
    out_ref[0] = s - _moving_avg_2d(s)


def _enc_ffn(x, a, lp):
    Bq, L, D = x.shape
    b2 = lambda v: v.reshape(1, -1)
    args = (x, a, lp['W1'], b2(lp['b1']), lp['W2'], b2(lp['b2']))
    in_specs = [_batch_spec(x.shape), _batch_spec(a.shape)] + \
               [_full_spec(w.shape) for w in args[2:]]
    return pl.pallas_call(
        _enc_ffn_body,
        grid=(Bq,),
        in_specs=in_specs,
        out_specs=_batch_spec((Bq, L, D)),
        out_shape=jax.ShapeDtypeStruct((Bq, L, D), jnp.float32),
    )(*args)


def _add_decomp_body(x_ref, a_ref, seas_ref, tr_ref):
    u = x_ref[0] + a_ref[0]
    ma = _moving_avg_2d(u)
    seas_ref[0] = u - ma
    tr_ref[0] = ma


def _add_decomp(x, a):
    Bq, L, D = x.shape
    return pl.pallas_call(
        _add_decomp_body,
        grid=(Bq,),
        in_specs=[_batch_spec(x.shape), _batch_spec(a.shape)],
        out_specs=[_batch_spec((Bq, L, D)), _batch_spec((Bq, L, D))],
        out_shape=[jax.ShapeDtypeStruct((Bq, L, D), jnp.float32),
                   jax.ShapeDtypeStruct((Bq, L, D), jnp.float32)],
    )(x, a)


def _dec_ffn_body(x_ref, w1_ref, b1_ref, w2_ref, b2_ref, seas_ref, tr_ref):
    x = x_ref[0]
    h = jax.nn.gelu(jnp.dot(x, w1_ref[...], precision=_PREC) + b1_ref[...])
    y = jnp.dot(h, w2_ref[...], precision=_PREC) + b2_ref[...]
    u = y + x
    ma = _moving_avg_2d(u)
    seas_ref[0] = u - ma
    tr_ref[0] = ma


def _dec_ffn(x, lp):
    Bq, L, D = x.shape
    b2 = lambda v: v.reshape(1, -1)
    args = (x, lp['W1'], b2(lp['b1']), lp['W2'], b2(lp['b2']))
    in_specs = [_batch_spec(x.shape)] + [_full_spec(w.shape) for w in args[1:]]
    return pl.pallas_call(
        _dec_ffn_body,
        grid=(Bq,),
        in_specs=in_specs,
        out_specs=[_batch_spec((Bq, L, D)), _batch_spec((Bq, L, D))],
        out_shape=[jax.ShapeDtypeStruct((Bq, L, D), jnp.float32),
                   jax.ShapeDtypeStruct((Bq, L, D), jnp.float32)],
    )(*args)


def _trend_body(t1_ref, t2_ref, t3_ref, w3_ref, out_ref):
    t = t1_ref[0] + t2_ref[0] + t3_ref[0]
    tm1 = jnp.concatenate([t[-1:], t[:-1]], axis=0)
    tp1 = jnp.concatenate([t[1:], t[:1]], axis=0)
    out_ref[0] = (jnp.dot(tm1, w3_ref[0], precision=_PREC)
                  + jnp.dot(t, w3_ref[1], precision=_PREC)
                  + jnp.dot(tp1, w3_ref[2], precision=_PREC))


def _trend_conv(t1, t2, t3, w3):
    Bq, L, _ = t1.shape
    return pl.pallas_call(
        _trend_body,
        grid=(Bq,),
        in_specs=[_batch_spec(t1.shape), _batch_spec(t2.shape),
                  _batch_spec(t3.shape), _full_spec(w3.shape)],
        out_specs=_batch_spec((Bq, L, _CIN)),
        out_shape=jax.ShapeDtypeStruct((Bq, L, _CIN), jnp.float32),
    )(t1, t2, t3, w3)


def _norm_body(x_ref, g_ref, b_ref, out_ref):
    x = x_ref[0]
    mu = jnp.mean(x, axis=-1, keepdims=True)
    var = jnp.mean((x - mu) ** 2, axis=-1, keepdims=True)
    xh = (x - mu) / jnp.sqrt(var + 1e-5) * g_ref[...] + b_ref[...]
    out_ref[0] = xh - jnp.mean(xh, axis=0, keepdims=True)


def _norm(x, g, b):
    Bq, L, D = x.shape
    g2, b2 = g.reshape(1, -1), b.reshape(1, -1)
    return pl.pallas_call(
        _norm_body,
        grid=(Bq,),
        in_specs=[_batch_spec(x.shape), _full_spec(g2.shape),
                  _full_spec(b2.shape)],
        out_specs=_batch_spec((Bq, L, D)),
        out_shape=jax.ShapeDtypeStruct((Bq, L, D), jnp.float32),
    )(x, g2, b2)


def _final_body(x_ref, g_ref, b_ref, wp_ref, bp_ref, ti_ref, tr_ref, out_ref):
    x = x_ref[0]
    mu = jnp.mean(x, axis=-1, keepdims=True)
    var = jnp.mean((x - mu) ** 2, axis=-1, keepdims=True)
    xh = (x - mu) / jnp.sqrt(var + 1e-5) * g_ref[...] + b_ref[...]
    xh = xh - jnp.mean(xh, axis=0, keepdims=True)
    seasonal = jnp.dot(xh, wp_ref[...], precision=_PREC) + bp_ref[...]
    full = seasonal + ti_ref[0] + tr_ref[0]
    out_ref[0] = full[_LABEL:, :]


def _final(x, g, b, wp, bp, ti, tr):
    Bq, L, D = x.shape
    g2, b2, bp2 = g.reshape(1, -1), b.reshape(1, -1), bp.reshape(1, -1)
    args = (x, g2, b2, wp, bp2, ti, tr)
    in_specs = [_batch_spec(x.shape), _full_spec(g2.shape),
                _full_spec(b2.shape), _full_spec(wp.shape),
                _full_spec(bp2.shape), _batch_spec(ti.shape),
                _batch_spec(tr.shape)]
    return pl.pallas_call(
        _final_body,
        grid=(Bq,),
        in_specs=in_specs,
        out_specs=_batch_spec((Bq, _PRED, _CIN)),
        out_shape=jax.ShapeDtypeStruct((Bq, _PRED, _CIN), jnp.float32),
    )(*args)


def kernel(x_enc, x_mark_enc, x_dec, x_mark_dec, params):
    p = params
    Bq = x_enc.shape[0]

    seas, ma, mean = _head_decomp(x_enc)
    trend_init = jnp.concatenate(
        [ma[:, -_LABEL:, :], jnp.repeat(mean, _PRED, axis=1)], axis=1)
    seasonal_init = jnp.concatenate(
        [seas[:, -_LABEL:, :],
         jnp.zeros((Bq, _PRED, _CIN), jnp.float32)], axis=1)

    enc = _embed(x_enc, x_mark_enc, p['tok_enc'], p['te_enc'],
                 _pos_encoding(_SEQ, _DM))
    for lp in p['enc_layers']:
        a = _attn(enc, enc, lp['attn'], _SEQ)
        enc = _enc_ffn(enc, a, lp)
    enc = _norm(enc, p['enc_norm_g'], p['enc_norm_b'])

    dec = _embed(seasonal_init, x_mark_dec, p['tok_dec'], p['te_dec'],
                 _pos_encoding(_DEC, _DM))
    enc_trunc = enc[:, :_DEC, :]
    trend_total = None
    for lp in p['dec_layers']:
        a1 = _attn(dec, dec, lp['self_attn'], _DEC)
        x1, t1 = _add_decomp(dec, a1)
        a2 = _attn(x1, enc_trunc, lp['cross_attn'], _DEC)
        x2, t2 = _add_decomp(x1, a2)
        dec, t3 = _dec_ffn(x2, lp)
        rt = _trend_conv(t1, t2, t3, lp['Wtrend'])
        trend_total = rt if trend_total is None else trend_total + rt

    return _final(dec, p['dec_norm_g'], p['dec_norm_b'], p['Wproj'],
                  p['bproj'], trend_init, trend_total)


# speedup vs baseline: 12.2365x; 1.5135x over previous
import math

import jax
import jax.numpy as jnp
import numpy as np
from jax.experimental import pallas as pl
from jax.experimental.pallas import tpu as pltpu

_B = 8
_SEQ = 512
_LABEL = 96
_PRED = 192
_DEC = _LABEL + _PRED
_CIN = 321
_DM = 512
_DFF = 2048
_MA = 25
_PAD = (_MA - 1) // 2
_FACTOR = 3

_PREC = jax.lax.Precision.HIGHEST


def _bmm(a, b):
    return jnp.dot(a.astype(jnp.bfloat16), b.astype(jnp.bfloat16),
                   preferred_element_type=jnp.float32)


def _pos_encoding(L, d):
    pe = np.zeros((L, d), dtype=np.float32)
    pos = np.arange(L, dtype=np.float32)[:, None]
    div = np.exp(np.arange(0, d, 2, dtype=np.float32) * (-math.log(10000.0) / d))
    pe[:, 0::2] = np.sin(pos * div)
    pe[:, 1::2] = np.cos(pos * div)
    return jnp.asarray(pe)


def _dft_mats(L):
    t = np.arange(L)
    ft = np.outer(t, t) % L
    ang = (2.0 * np.pi / L) * ft.astype(np.float64)
    return (jnp.asarray(np.cos(ang), dtype=jnp.float32),
            jnp.asarray(np.sin(ang), dtype=jnp.float32))


def _batch_spec(shape):
    nd = len(shape)
    return pl.BlockSpec((1,) + tuple(shape[1:]),
                        lambda b, _nd=nd: (b,) + (0,) * (_nd - 1))


def _full_spec(shape):
    nd = len(shape)
    return pl.BlockSpec(tuple(shape), lambda b, _nd=nd: (0,) * _nd)


def _moving_avg_2d(x):
    L, C = x.shape
    front = jnp.broadcast_to(x[0:1], (_PAD, C))
    end = jnp.broadcast_to(x[L - 1:L], (_PAD, C))
    xp = jnp.concatenate([front, x, end], axis=0)
    acc = xp[0:L]
    for j in range(1, _MA):
        acc = acc + xp[j:j + L]
    return acc * (1.0 / _MA)


def _head_body(x_ref, seas_ref, ma_ref, mean_ref):
    x = x_ref[0]
    ma = _moving_avg_2d(x)
    seas_ref[0] = x - ma
    ma_ref[0] = ma
    mean_ref[0] = jnp.mean(x, axis=0, keepdims=True)


def _head_decomp(x):
    Bq, L, C = x.shape
    return pl.pallas_call(
        _head_body,
        grid=(Bq,),
        in_specs=[_batch_spec(x.shape)],
        out_specs=[_batch_spec((Bq, L, C)), _batch_spec((Bq, L, C)),
                   _batch_spec((Bq, 1, C))],
        out_shape=[jax.ShapeDtypeStruct((Bq, L, C), jnp.float32),
                   jax.ShapeDtypeStruct((Bq, L, C), jnp.float32),
                   jax.ShapeDtypeStruct((Bq, 1, C), jnp.float32)],
    )(x)


def _embed_body(x_ref, mark_ref, w3_ref, te_ref, pe_ref, out_ref):
    x = x_ref[0]
    xm1 = jnp.concatenate([x[-1:], x[:-1]], axis=0)
    xp1 = jnp.concatenate([x[1:], x[:1]], axis=0)
    acc = _bmm(xm1, w3_ref[0])
    acc = acc + _bmm(x, w3_ref[1])
    acc = acc + _bmm(xp1, w3_ref[2])
    acc = acc + _bmm(mark_ref[0], te_ref[...])
    out_ref[0] = acc + pe_ref[...]


def _embed(x, mark, w3, te, pe):
    Bq, L, _ = x.shape
    return pl.pallas_call(
        _embed_body,
        grid=(Bq,),
        in_specs=[_batch_spec(x.shape), _batch_spec(mark.shape),
                  _full_spec(w3.shape), _full_spec(te.shape),
                  _full_spec(pe.shape)],
        out_specs=_batch_spec((Bq, L, _DM)),
        out_shape=jax.ShapeDtypeStruct((Bq, L, _DM), jnp.float32),
    )(x, mark, w3, te, pe)


def _make_attn_body(L, topk, scale):
    def body(xq_ref, xkv_ref, wq_ref, bq_ref, wk_ref, bk_ref, wv_ref, bv_ref,
             wo_ref, bo_ref, cf_ref, sf_ref, out_ref):
        xq = xq_ref[0]
        xkv = xkv_ref[0]
        q = _bmm(xq, wq_ref[...]) + bq_ref[...]
        k = _bmm(xkv, wk_ref[...]) + bk_ref[...]
        v = _bmm(xkv, wv_ref[...]) + bv_ref[...]

        cf = cf_ref[...]
        sf = sf_ref[...]
        qr = jnp.dot(cf, q, precision=_PREC)
        qi = -jnp.dot(sf, q, precision=_PREC)
        kr = jnp.dot(cf, k, precision=_PREC)
        ki = -jnp.dot(sf, k, precision=_PREC)
        pr = jnp.sum(qr * kr + qi * ki, axis=1, keepdims=True)
        pi = jnp.sum(qi * kr - qr * ki, axis=1, keepdims=True)
        mv = (jnp.dot(cf, pr, precision=_PREC)
              - jnp.dot(sf, pi, precision=_PREC)) * scale

        iota = jax.lax.broadcasted_iota(jnp.int32, (L, 1), 0)
        work = mv
        sel = jnp.zeros((L, 1), jnp.bool_)
        m0 = None
        for i in range(topk):
            m = jnp.max(work)
            if i == 0:
                m0 = m
            idx = jnp.min(jnp.where(work == m, iota, L))
            onehot = iota == idx
            sel = jnp.logical_or(sel, onehot)
            work = jnp.where(onehot, -jnp.inf, work)

        e = jnp.where(sel, jnp.exp(mv - m0), 0.0)
        cvec = e / jnp.sum(e)

        cr = jnp.dot(cf, cvec, precision=_PREC)
        ci = -jnp.dot(sf, cvec, precision=_PREC)
        vr = jnp.dot(cf, v, precision=_PREC)
        vi = -jnp.dot(sf, v, precision=_PREC)
        re = cr * vr + ci * vi
        im = cr * vi - ci * vr
        agg = (jnp.dot(cf, re, precision=_PREC)
               - jnp.dot(sf, im, precision=_PREC)) * (1.0 / L)

        out_ref[0] = _bmm(agg, wo_ref[...]) + bo_ref[...]

    return body


def _attn(xq, xkv, ap, L):
    topk = int(_FACTOR * math.log(L))
    scale = 1.0 / (L * _DM)
    cf, sf = _dft_mats(L)
    b2 = lambda v: v.reshape(1, -1)
    args = (xq, xkv, ap['Wq'], b2(ap['bq']), ap['Wk'], b2(ap['bk']),
            ap['Wv'], b2(ap['bv']), ap['Wo'], b2(ap['bo']), cf, sf)
    in_specs = [_batch_spec(xq.shape), _batch_spec(xkv.shape)] + \
               [_full_spec(a.shape) for a in args[2:]]
    return pl.pallas_call(
        _make_attn_body(L, topk, scale),
        grid=(xq.shape[0],),
        in_specs=in_specs,
        out_specs=_batch_spec((xq.shape[0], L, _DM)),
        out_shape=jax.ShapeDtypeStruct((xq.shape[0], L, _DM), jnp.float32),
    )(*args)


def _enc_ffn_body(x_ref, a_ref, w1_ref, b1_ref, w2_ref, b2_ref, out_ref):
    u = x_ref[0] + a_ref[0]
    x1 = u - _moving_avg_2d(u)
    h = jax.nn.gelu(_bmm(x1, w1_ref[...]) + b1_ref[...])
    y = _bmm(h, w2_ref[...]) + b2_ref[...]
    s = x1 + y
    out_ref[0] = s - _moving_avg_2d(s)


def _enc_ffn(x, a, lp):
    Bq, L, D = x.shape
    b2 = lambda v: v.reshape(1, -1)
    args = (x, a, lp['W1'], b2(lp['b1']), lp['W2'], b2(lp['b2']))
    in_specs = [_batch_spec(x.shape), _batch_spec(a.shape)] + \
               [_full_spec(w.shape) for w in args[2:]]
    return pl.pallas_call(
        _enc_ffn_body,
        grid=(Bq,),
        in_specs=in_specs,
        out_specs=_batch_spec((Bq, L, D)),
        out_shape=jax.ShapeDtypeStruct((Bq, L, D), jnp.float32),
    )(*args)


def _add_decomp_body(x_ref, a_ref, seas_ref, tr_ref):
    u = x_ref[0] + a_ref[0]
    ma = _moving_avg_2d(u)
    seas_ref[0] = u - ma
    tr_ref[0] = ma


def _add_decomp(x, a):
    Bq, L, D = x.shape
    return pl.pallas_call(
        _add_decomp_body,
        grid=(Bq,),
        in_specs=[_batch_spec(x.shape), _batch_spec(a.shape)],
        out_specs=[_batch_spec((Bq, L, D)), _batch_spec((Bq, L, D))],
        out_shape=[jax.ShapeDtypeStruct((Bq, L, D), jnp.float32),
                   jax.ShapeDtypeStruct((Bq, L, D), jnp.float32)],
    )(x, a)


def _dec_ffn_body(x_ref, w1_ref, b1_ref, w2_ref, b2_ref, seas_ref, tr_ref):
    x = x_ref[0]
    h = jax.nn.gelu(_bmm(x, w1_ref[...]) + b1_ref[...])
    y = _bmm(h, w2_ref[...]) + b2_ref[...]
    u = y + x
    ma = _moving_avg_2d(u)
    seas_ref[0] = u - ma
    tr_ref[0] = ma


def _dec_ffn(x, lp):
    Bq, L, D = x.shape
    b2 = lambda v: v.reshape(1, -1)
    args = (x, lp['W1'], b2(lp['b1']), lp['W2'], b2(lp['b2']))
    in_specs = [_batch_spec(x.shape)] + [_full_spec(w.shape) for w in args[1:]]
    return pl.pallas_call(
        _dec_ffn_body,
        grid=(Bq,),
        in_specs=in_specs,
        out_specs=[_batch_spec((Bq, L, D)), _batch_spec((Bq, L, D))],
        out_shape=[jax.ShapeDtypeStruct((Bq, L, D), jnp.float32),
                   jax.ShapeDtypeStruct((Bq, L, D), jnp.float32)],
    )(*args)


def _trend_body(t1_ref, t2_ref, t3_ref, w3_ref, out_ref):
    t = t1_ref[0] + t2_ref[0] + t3_ref[0]
    tm1 = jnp.concatenate([t[-1:], t[:-1]], axis=0)
    tp1 = jnp.concatenate([t[1:], t[:1]], axis=0)
    out_ref[0] = (_bmm(tm1, w3_ref[0]) + _bmm(t, w3_ref[1])
                  + _bmm(tp1, w3_ref[2]))


def _trend_conv(t1, t2, t3, w3):
    Bq, L, _ = t1.shape
    return pl.pallas_call(
        _trend_body,
        grid=(Bq,),
        in_specs=[_batch_spec(t1.shape), _batch_spec(t2.shape),
                  _batch_spec(t3.shape), _full_spec(w3.shape)],
        out_specs=_batch_spec((Bq, L, _CIN)),
        out_shape=jax.ShapeDtypeStruct((Bq, L, _CIN), jnp.float32),
    )(t1, t2, t3, w3)


def _norm_body(x_ref, g_ref, b_ref, out_ref):
    x = x_ref[0]
    mu = jnp.mean(x, axis=-1, keepdims=True)
    var = jnp.mean((x - mu) ** 2, axis=-1, keepdims=True)
    xh = (x - mu) / jnp.sqrt(var + 1e-5) * g_ref[...] + b_ref[...]
    out_ref[0] = xh - jnp.mean(xh, axis=0, keepdims=True)


def _norm(x, g, b):
    Bq, L, D = x.shape
    g2, b2 = g.reshape(1, -1), b.reshape(1, -1)
    return pl.pallas_call(
        _norm_body,
        grid=(Bq,),
        in_specs=[_batch_spec(x.shape), _full_spec(g2.shape),
                  _full_spec(b2.shape)],
        out_specs=_batch_spec((Bq, L, D)),
        out_shape=jax.ShapeDtypeStruct((Bq, L, D), jnp.float32),
    )(x, g2, b2)


def _final_body(x_ref, g_ref, b_ref, wp_ref, bp_ref, ti_ref, tr_ref, out_ref):
    x = x_ref[0]
    mu = jnp.mean(x, axis=-1, keepdims=True)
    var = jnp.mean((x - mu) ** 2, axis=-1, keepdims=True)
    xh = (x - mu) / jnp.sqrt(var + 1e-5) * g_ref[...] + b_ref[...]
    xh = xh - jnp.mean(xh, axis=0, keepdims=True)
    seasonal = _bmm(xh, wp_ref[...]) + bp_ref[...]
    full = seasonal + ti_ref[0] + tr_ref[0]
    out_ref[0] = full[_LABEL:, :]


def _final(x, g, b, wp, bp, ti, tr):
    Bq, L, D = x.shape
    g2, b2, bp2 = g.reshape(1, -1), b.reshape(1, -1), bp.reshape(1, -1)
    args = (x, g2, b2, wp, bp2, ti, tr)
    in_specs = [_batch_spec(x.shape), _full_spec(g2.shape),
                _full_spec(b2.shape), _full_spec(wp.shape),
                _full_spec(bp2.shape), _batch_spec(ti.shape),
                _batch_spec(tr.shape)]
    return pl.pallas_call(
        _final_body,
        grid=(Bq,),
        in_specs=in_specs,
        out_specs=_batch_spec((Bq, _PRED, _CIN)),
        out_shape=jax.ShapeDtypeStruct((Bq, _PRED, _CIN), jnp.float32),
    )(*args)


def kernel(x_enc, x_mark_enc, x_dec, x_mark_dec, params):
    p = params
    Bq = x_enc.shape[0]

    seas, ma, mean = _head_decomp(x_enc)
    trend_init = jnp.concatenate(
        [ma[:, -_LABEL:, :], jnp.repeat(mean, _PRED, axis=1)], axis=1)
    seasonal_init = jnp.concatenate(
        [seas[:, -_LABEL:, :],
         jnp.zeros((Bq, _PRED, _CIN), jnp.float32)], axis=1)

    enc = _embed(x_enc, x_mark_enc, p['tok_enc'], p['te_enc'],
                 _pos_encoding(_SEQ, _DM))
    for lp in p['enc_layers']:
        a = _attn(enc, enc, lp['attn'], _SEQ)
        enc = _enc_ffn(enc, a, lp)
    enc = _norm(enc, p['enc_norm_g'], p['enc_norm_b'])

    dec = _embed(seasonal_init, x_mark_dec, p['tok_dec'], p['te_dec'],
                 _pos_encoding(_DEC, _DM))
    enc_trunc = enc[:, :_DEC, :]
    trend_total = None
    for lp in p['dec_layers']:
        a1 = _attn(dec, dec, lp['self_attn'], _DEC)
        x1, t1 = _add_decomp(dec, a1)
        a2 = _attn(x1, enc_trunc, lp['cross_attn'], _DEC)
        x2, t2 = _add_decomp(x1, a2)
        dec, t3 = _dec_ffn(x2, lp)
        rt = _trend_conv(t1, t2, t3, lp['Wtrend'])
        trend_total = rt if trend_total is None else trend_total + rt

    return _final(dec, p['dec_norm_g'], p['dec_norm_b'], p['Wproj'],
                  p['bproj'], trend_init, trend_total)


# speedup vs baseline: 12.5991x; 1.0296x over previous
import math

import jax
import jax.numpy as jnp
import numpy as np
from jax.experimental import pallas as pl
from jax.experimental.pallas import tpu as pltpu

_B = 8
_SEQ = 512
_LABEL = 96
_PRED = 192
_DEC = _LABEL + _PRED
_CIN = 321
_DM = 512
_DFF = 2048
_MA = 25
_PAD = (_MA - 1) // 2
_FACTOR = 3

_PREC = jax.lax.Precision.HIGHEST


def _bmm(a, b):
    return jnp.dot(a.astype(jnp.bfloat16), b.astype(jnp.bfloat16),
                   preferred_element_type=jnp.float32)


def _pos_encoding(L, d):
    pe = np.zeros((L, d), dtype=np.float32)
    pos = np.arange(L, dtype=np.float32)[:, None]
    div = np.exp(np.arange(0, d, 2, dtype=np.float32) * (-math.log(10000.0) / d))
    pe[:, 0::2] = np.sin(pos * div)
    pe[:, 1::2] = np.cos(pos * div)
    return jnp.asarray(pe)


def _dft_mats(L):
    t = np.arange(L)
    ft = np.outer(t, t) % L
    ang = (2.0 * np.pi / L) * ft.astype(np.float64)
    return (jnp.asarray(np.cos(ang), dtype=jnp.float32),
            jnp.asarray(np.sin(ang), dtype=jnp.float32))


def _batch_spec(shape):
    nd = len(shape)
    return pl.BlockSpec((1,) + tuple(shape[1:]),
                        lambda b, _nd=nd: (b,) + (0,) * (_nd - 1))


def _full_spec(shape):
    nd = len(shape)
    return pl.BlockSpec(tuple(shape), lambda b, _nd=nd: (0,) * _nd)


def _ma_mat(L):
    A = np.zeros((L, L), np.float32)
    for t in range(L):
        for j in range(t - _PAD, t + _PAD + 1):
            A[t, min(max(j, 0), L - 1)] += 1.0
    return jnp.asarray(A, jnp.bfloat16)


def _ma_mm(u, A):
    hi = u.astype(jnp.bfloat16)
    r1 = u - hi.astype(jnp.float32)
    mid = r1.astype(jnp.bfloat16)
    lo = (r1 - mid.astype(jnp.float32)).astype(jnp.bfloat16)
    s = (jnp.dot(A, hi, preferred_element_type=jnp.float32)
         + jnp.dot(A, mid, preferred_element_type=jnp.float32)
         + jnp.dot(A, lo, preferred_element_type=jnp.float32))
    return s * (1.0 / _MA)


def _head_body(x_ref, A_ref, seas_ref, ma_ref, mean_ref):
    x = x_ref[0]
    ma = _ma_mm(x, A_ref[...])
    seas_ref[0] = x - ma
    ma_ref[0] = ma
    mean_ref[0] = jnp.mean(x, axis=0, keepdims=True)


def _head_decomp(x):
    Bq, L, C = x.shape
    A = _ma_mat(L)
    return pl.pallas_call(
        _head_body,
        grid=(Bq,),
        in_specs=[_batch_spec(x.shape), _full_spec(A.shape)],
        out_specs=[_batch_spec((Bq, L, C)), _batch_spec((Bq, L, C)),
                   _batch_spec((Bq, 1, C))],
        out_shape=[jax.ShapeDtypeStruct((Bq, L, C), jnp.float32),
                   jax.ShapeDtypeStruct((Bq, L, C), jnp.float32),
                   jax.ShapeDtypeStruct((Bq, 1, C), jnp.float32)],
    )(x, A)


def _embed_body(x_ref, mark_ref, w3_ref, te_ref, pe_ref, out_ref):
    x = x_ref[0]
    xm1 = jnp.concatenate([x[-1:], x[:-1]], axis=0)
    xp1 = jnp.concatenate([x[1:], x[:1]], axis=0)
    acc = _bmm(xm1, w3_ref[0])
    acc = acc + _bmm(x, w3_ref[1])
    acc = acc + _bmm(xp1, w3_ref[2])
    acc = acc + _bmm(mark_ref[0], te_ref[...])
    out_ref[0] = acc + pe_ref[...]


def _embed(x, mark, w3, te, pe):
    Bq, L, _ = x.shape
    return pl.pallas_call(
        _embed_body,
        grid=(Bq,),
        in_specs=[_batch_spec(x.shape), _batch_spec(mark.shape),
                  _full_spec(w3.shape), _full_spec(te.shape),
                  _full_spec(pe.shape)],
        out_specs=_batch_spec((Bq, L, _DM)),
        out_shape=jax.ShapeDtypeStruct((Bq, L, _DM), jnp.float32),
    )(x, mark, w3, te, pe)


def _make_attn_body(L, topk, scale):
    def body(xq_ref, xkv_ref, wq_ref, bq_ref, wk_ref, bk_ref, wv_ref, bv_ref,
             wo_ref, bo_ref, cf_ref, sf_ref, out_ref):
        xq = xq_ref[0]
        xkv = xkv_ref[0]
        q = _bmm(xq, wq_ref[...]) + bq_ref[...]
        k = _bmm(xkv, wk_ref[...]) + bk_ref[...]
        v = _bmm(xkv, wv_ref[...]) + bv_ref[...]

        cf = cf_ref[...]
        sf = sf_ref[...]
        qr = jnp.dot(cf, q, precision=_PREC)
        qi = -jnp.dot(sf, q, precision=_PREC)
        kr = jnp.dot(cf, k, precision=_PREC)
        ki = -jnp.dot(sf, k, precision=_PREC)
        pr = jnp.sum(qr * kr + qi * ki, axis=1, keepdims=True)
        pi = jnp.sum(qi * kr - qr * ki, axis=1, keepdims=True)
        mv = (jnp.dot(cf, pr, precision=_PREC)
              - jnp.dot(sf, pi, precision=_PREC)) * scale

        iota = jax.lax.broadcasted_iota(jnp.int32, (L, 1), 0)
        work = mv
        sel = jnp.zeros((L, 1), jnp.bool_)
        m0 = None
        for i in range(topk):
            m = jnp.max(work)
            if i == 0:
                m0 = m
            idx = jnp.min(jnp.where(work == m, iota, L))
            onehot = iota == idx
            sel = jnp.logical_or(sel, onehot)
            work = jnp.where(onehot, -jnp.inf, work)

        e = jnp.where(sel, jnp.exp(mv - m0), 0.0)
        cvec = e / jnp.sum(e)

        cr = jnp.dot(cf, cvec, precision=_PREC)
        ci = -jnp.dot(sf, cvec, precision=_PREC)
        vr = jnp.dot(cf, v, precision=_PREC)
        vi = -jnp.dot(sf, v, precision=_PREC)
        re = cr * vr + ci * vi
        im = cr * vi - ci * vr
        agg = (jnp.dot(cf, re, precision=_PREC)
               - jnp.dot(sf, im, precision=_PREC)) * (1.0 / L)

        out_ref[0] = _bmm(agg, wo_ref[...]) + bo_ref[...]

    return body


def _attn(xq, xkv, ap, L):
    topk = int(_FACTOR * math.log(L))
    scale = 1.0 / (L * _DM)
    cf, sf = _dft_mats(L)
    b2 = lambda v: v.reshape(1, -1)
    args = (xq, xkv, ap['Wq'], b2(ap['bq']), ap['Wk'], b2(ap['bk']),
            ap['Wv'], b2(ap['bv']), ap['Wo'], b2(ap['bo']), cf, sf)
    in_specs = [_batch_spec(xq.shape), _batch_spec(xkv.shape)] + \
               [_full_spec(a.shape) for a in args[2:]]
    return pl.pallas_call(
        _make_attn_body(L, topk, scale),
        grid=(xq.shape[0],),
        in_specs=in_specs,
        out_specs=_batch_spec((xq.shape[0], L, _DM)),
        out_shape=jax.ShapeDtypeStruct((xq.shape[0], L, _DM), jnp.float32),
    )(*args)


def _enc_ffn_body(x_ref, a_ref, w1_ref, b1_ref, w2_ref, b2_ref, A_ref,
                  out_ref):
    u = x_ref[0] + a_ref[0]
    x1 = u - _ma_mm(u, A_ref[...])
    h = jax.nn.gelu(_bmm(x1, w1_ref[...]) + b1_ref[...])
    y = _bmm(h, w2_ref[...]) + b2_ref[...]
    s = x1 + y
    out_ref[0] = s - _ma_mm(s, A_ref[...])


def _enc_ffn(x, a, lp):
    Bq, L, D = x.shape
    b2 = lambda v: v.reshape(1, -1)
    args = (x, a, lp['W1'], b2(lp['b1']), lp['W2'], b2(lp['b2']), _ma_mat(L))
    in_specs = [_batch_spec(x.shape), _batch_spec(a.shape)] + \
               [_full_spec(w.shape) for w in args[2:]]
    return pl.pallas_call(
        _enc_ffn_body,
        grid=(Bq,),
        in_specs=in_specs,
        out_specs=_batch_spec((Bq, L, D)),
        out_shape=jax.ShapeDtypeStruct((Bq, L, D), jnp.float32),
    )(*args)


def _add_decomp_body(x_ref, a_ref, A_ref, seas_ref, tr_ref):
    u = x_ref[0] + a_ref[0]
    ma = _ma_mm(u, A_ref[...])
    seas_ref[0] = u - ma
    tr_ref[0] = ma


def _add_decomp(x, a):
    Bq, L, D = x.shape
    A = _ma_mat(L)
    return pl.pallas_call(
        _add_decomp_body,
        grid=(Bq,),
        in_specs=[_batch_spec(x.shape), _batch_spec(a.shape),
                  _full_spec(A.shape)],
        out_specs=[_batch_spec((Bq, L, D)), _batch_spec((Bq, L, D))],
        out_shape=[jax.ShapeDtypeStruct((Bq, L, D), jnp.float32),
                   jax.ShapeDtypeStruct((Bq, L, D), jnp.float32)],
    )(x, a, A)


def _dec_ffn_body(x_ref, w1_ref, b1_ref, w2_ref, b2_ref, A_ref, seas_ref,
                  tr_ref):
    x = x_ref[0]
    h = jax.nn.gelu(_bmm(x, w1_ref[...]) + b1_ref[...])
    y = _bmm(h, w2_ref[...]) + b2_ref[...]
    u = y + x
    ma = _ma_mm(u, A_ref[...])
    seas_ref[0] = u - ma
    tr_ref[0] = ma


def _dec_ffn(x, lp):
    Bq, L, D = x.shape
    b2 = lambda v: v.reshape(1, -1)
    args = (x, lp['W1'], b2(lp['b1']), lp['W2'], b2(lp['b2']), _ma_mat(L))
    in_specs = [_batch_spec(x.shape)] + [_full_spec(w.shape) for w in args[1:]]
    return pl.pallas_call(
        _dec_ffn_body,
        grid=(Bq,),
        in_specs=in_specs,
        out_specs=[_batch_spec((Bq, L, D)), _batch_spec((Bq, L, D))],
        out_shape=[jax.ShapeDtypeStruct((Bq, L, D), jnp.float32),
                   jax.ShapeDtypeStruct((Bq, L, D), jnp.float32)],
    )(*args)


def _trend_body(t1_ref, t2_ref, t3_ref, w3_ref, out_ref):
    t = t1_ref[0] + t2_ref[0] + t3_ref[0]
    tm1 = jnp.concatenate([t[-1:], t[:-1]], axis=0)
    tp1 = jnp.concatenate([t[1:], t[:1]], axis=0)
    out_ref[0] = (_bmm(tm1, w3_ref[0]) + _bmm(t, w3_ref[1])
                  + _bmm(tp1, w3_ref[2]))


def _trend_conv(t1, t2, t3, w3):
    Bq, L, _ = t1.shape
    return pl.pallas_call(
        _trend_body,
        grid=(Bq,),
        in_specs=[_batch_spec(t1.shape), _batch_spec(t2.shape),
                  _batch_spec(t3.shape), _full_spec(w3.shape)],
        out_specs=_batch_spec((Bq, L, _CIN)),
        out_shape=jax.ShapeDtypeStruct((Bq, L, _CIN), jnp.float32),
    )(t1, t2, t3, w3)


def _norm_body(x_ref, g_ref, b_ref, out_ref):
    x = x_ref[0]
    mu = jnp.mean(x, axis=-1, keepdims=True)
    var = jnp.mean((x - mu) ** 2, axis=-1, keepdims=True)
    xh = (x - mu) / jnp.sqrt(var + 1e-5) * g_ref[...] + b_ref[...]
    out_ref[0] = xh - jnp.mean(xh, axis=0, keepdims=True)


def _norm(x, g, b):
    Bq, L, D = x.shape
    g2, b2 = g.reshape(1, -1), b.reshape(1, -1)
    return pl.pallas_call(
        _norm_body,
        grid=(Bq,),
        in_specs=[_batch_spec(x.shape), _full_spec(g2.shape),
                  _full_spec(b2.shape)],
        out_specs=_batch_spec((Bq, L, D)),
        out_shape=jax.ShapeDtypeStruct((Bq, L, D), jnp.float32),
    )(x, g2, b2)


def _final_body(x_ref, g_ref, b_ref, wp_ref, bp_ref, ti_ref, tr_ref, out_ref):
    x = x_ref[0]
    mu = jnp.mean(x, axis=-1, keepdims=True)
    var = jnp.mean((x - mu) ** 2, axis=-1, keepdims=True)
    xh = (x - mu) / jnp.sqrt(var + 1e-5) * g_ref[...] + b_ref[...]
    xh = xh - jnp.mean(xh, axis=0, keepdims=True)
    seasonal = _bmm(xh, wp_ref[...]) + bp_ref[...]
    full = seasonal + ti_ref[0] + tr_ref[0]
    out_ref[0] = full[_LABEL:, :]


def _final(x, g, b, wp, bp, ti, tr):
    Bq, L, D = x.shape
    g2, b2, bp2 = g.reshape(1, -1), b.reshape(1, -1), bp.reshape(1, -1)
    args = (x, g2, b2, wp, bp2, ti, tr)
    in_specs = [_batch_spec(x.shape), _full_spec(g2.shape),
                _full_spec(b2.shape), _full_spec(wp.shape),
                _full_spec(bp2.shape), _batch_spec(ti.shape),
                _batch_spec(tr.shape)]
    return pl.pallas_call(
        _final_body,
        grid=(Bq,),
        in_specs=in_specs,
        out_specs=_batch_spec((Bq, _PRED, _CIN)),
        out_shape=jax.ShapeDtypeStruct((Bq, _PRED, _CIN), jnp.float32),
    )(*args)


def kernel(x_enc, x_mark_enc, x_dec, x_mark_dec, params):
    p = params
    Bq = x_enc.shape[0]

    seas, ma, mean = _head_decomp(x_enc)
    trend_init = jnp.concatenate(
        [ma[:, -_LABEL:, :], jnp.repeat(mean, _PRED, axis=1)], axis=1)
    seasonal_init = jnp.concatenate(
        [seas[:, -_LABEL:, :],
         jnp.zeros((Bq, _PRED, _CIN), jnp.float32)], axis=1)

    enc = _embed(x_enc, x_mark_enc, p['tok_enc'], p['te_enc'],
                 _pos_encoding(_SEQ, _DM))
    for lp in p['enc_layers']:
        a = _attn(enc, enc, lp['attn'], _SEQ)
        enc = _enc_ffn(enc, a, lp)
    enc = _norm(enc, p['enc_norm_g'], p['enc_norm_b'])

    dec = _embed(seasonal_init, x_mark_dec, p['tok_dec'], p['te_dec'],
                 _pos_encoding(_DEC, _DM))
    enc_trunc = enc[:, :_DEC, :]
    trend_total = None
    for lp in p['dec_layers']:
        a1 = _attn(dec, dec, lp['self_attn'], _DEC)
        x1, t1 = _add_decomp(dec, a1)
        a2 = _attn(x1, enc_trunc, lp['cross_attn'], _DEC)
        x2, t2 = _add_decomp(x1, a2)
        dec, t3 = _dec_ffn(x2, lp)
        rt = _trend_conv(t1, t2, t3, lp['Wtrend'])
        trend_total = rt if trend_total is None else trend_total + rt

    return _final(dec, p['dec_norm_g'], p['dec_norm_b'], p['Wproj'],
                  p['bproj'], trend_init, trend_total)


# speedup vs baseline: 13.9774x; 1.1094x over previous
import math

import jax
import jax.numpy as jnp
import numpy as np
from jax.experimental import pallas as pl
from jax.experimental.pallas import tpu as pltpu

_B = 8
_SEQ = 512
_LABEL = 96
_PRED = 192
_DEC = _LABEL + _PRED
_CIN = 321
_DM = 512
_DFF = 2048
_MA = 25
_PAD = (_MA - 1) // 2
_FACTOR = 3

_PREC = jax.lax.Precision.HIGHEST


def _bmm(a, b):
    return jnp.dot(a.astype(jnp.bfloat16), b.astype(jnp.bfloat16),
                   preferred_element_type=jnp.float32)


def _pos_encoding(L, d):
    pe = np.zeros((L, d), dtype=np.float32)
    pos = np.arange(L, dtype=np.float32)[:, None]
    div = np.exp(np.arange(0, d, 2, dtype=np.float32) * (-math.log(10000.0) / d))
    pe[:, 0::2] = np.sin(pos * div)
    pe[:, 1::2] = np.cos(pos * div)
    return jnp.asarray(pe)


def _dft_half_mats(L):
    F = L // 2 + 1
    Fp = ((F + 7) // 8) * 8
    f = np.arange(Fp)
    t = np.arange(L)
    ft = np.outer(f, t) % L
    ang = (2.0 * np.pi / L) * ft.astype(np.float64)
    cf = np.cos(ang)
    sf = np.sin(ang)
    cf[F:] = 0.0
    sf[F:] = 0.0
    alpha = np.full((Fp, 1), 2.0)
    alpha[0] = 1.0
    alpha[F - 1] = 1.0 if L % 2 == 0 else 2.0
    alpha[F:] = 0.0
    ci = (alpha * cf).T
    si = (alpha * sf).T
    return (jnp.asarray(cf, dtype=jnp.float32),
            jnp.asarray(sf, dtype=jnp.float32),
            jnp.asarray(ci, dtype=jnp.float32),
            jnp.asarray(si, dtype=jnp.float32))


def _batch_spec(shape):
    nd = len(shape)
    return pl.BlockSpec((1,) + tuple(shape[1:]),
                        lambda b, _nd=nd: (b,) + (0,) * (_nd - 1))


def _full_spec(shape):
    nd = len(shape)
    return pl.BlockSpec(tuple(shape), lambda b, _nd=nd: (0,) * _nd)


def _ma_mat(L):
    A = np.zeros((L, L), np.float32)
    for t in range(L):
        for j in range(t - _PAD, t + _PAD + 1):
            A[t, min(max(j, 0), L - 1)] += 1.0
    return jnp.asarray(A, jnp.bfloat16)


def _ma_mm(u, A):
    hi = u.astype(jnp.bfloat16)
    r1 = u - hi.astype(jnp.float32)
    mid = r1.astype(jnp.bfloat16)
    lo = (r1 - mid.astype(jnp.float32)).astype(jnp.bfloat16)
    s = (jnp.dot(A, hi, preferred_element_type=jnp.float32)
         + jnp.dot(A, mid, preferred_element_type=jnp.float32)
         + jnp.dot(A, lo, preferred_element_type=jnp.float32))
    return s * (1.0 / _MA)


def _head_body(x_ref, A_ref, seas_ref, ma_ref, mean_ref):
    x = x_ref[0]
    ma = _ma_mm(x, A_ref[...])
    seas_ref[0] = x - ma
    ma_ref[0] = ma
    mean_ref[0] = jnp.mean(x, axis=0, keepdims=True)


def _head_decomp(x):
    Bq, L, C = x.shape
    A = _ma_mat(L)
    return pl.pallas_call(
        _head_body,
        grid=(Bq,),
        in_specs=[_batch_spec(x.shape), _full_spec(A.shape)],
        out_specs=[_batch_spec((Bq, L, C)), _batch_spec((Bq, L, C)),
                   _batch_spec((Bq, 1, C))],
        out_shape=[jax.ShapeDtypeStruct((Bq, L, C), jnp.float32),
                   jax.ShapeDtypeStruct((Bq, L, C), jnp.float32),
                   jax.ShapeDtypeStruct((Bq, 1, C), jnp.float32)],
    )(x, A)


def _embed_body(x_ref, mark_ref, w3_ref, te_ref, pe_ref, out_ref):
    x = x_ref[0]
    xm1 = jnp.concatenate([x[-1:], x[:-1]], axis=0)
    xp1 = jnp.concatenate([x[1:], x[:1]], axis=0)
    acc = _bmm(xm1, w3_ref[0])
    acc = acc + _bmm(x, w3_ref[1])
    acc = acc + _bmm(xp1, w3_ref[2])
    acc = acc + _bmm(mark_ref[0], te_ref[...])
    out_ref[0] = acc + pe_ref[...]


def _embed(x, mark, w3, te, pe):
    Bq, L, _ = x.shape
    return pl.pallas_call(
        _embed_body,
        grid=(Bq,),
        in_specs=[_batch_spec(x.shape), _batch_spec(mark.shape),
                  _full_spec(w3.shape), _full_spec(te.shape),
                  _full_spec(pe.shape)],
        out_specs=_batch_spec((Bq, L, _DM)),
        out_shape=jax.ShapeDtypeStruct((Bq, L, _DM), jnp.float32),
    )(x, mark, w3, te, pe)


def _make_attn_body(L, topk, scale):
    def body(xq_ref, xkv_ref, wq_ref, bq_ref, wk_ref, bk_ref, wv_ref, bv_ref,
             wo_ref, bo_ref, cf_ref, sf_ref, ci_ref, si_ref, out_ref):
        xq = xq_ref[0]
        xkv = xkv_ref[0]
        q = _bmm(xq, wq_ref[...]) + bq_ref[...]
        k = _bmm(xkv, wk_ref[...]) + bk_ref[...]
        v = _bmm(xkv, wv_ref[...]) + bv_ref[...]

        cf = cf_ref[...]
        sf = sf_ref[...]
        qr = jnp.dot(cf, q, precision=_PREC)
        qi = -jnp.dot(sf, q, precision=_PREC)
        kr = jnp.dot(cf, k, precision=_PREC)
        ki = -jnp.dot(sf, k, precision=_PREC)
        pr = jnp.sum(qr * kr + qi * ki, axis=1, keepdims=True)
        pi = jnp.sum(qi * kr - qr * ki, axis=1, keepdims=True)
        mv = (jnp.dot(ci_ref[...], pr, precision=_PREC)
              - jnp.dot(si_ref[...], pi, precision=_PREC)) * scale

        iota = jax.lax.broadcasted_iota(jnp.int32, (L, 1), 0)
        work = mv
        masks = []
        delays = []
        m0 = None
        for i in range(topk):
            m = jnp.max(work)
            if i == 0:
                m0 = m
            idx = jnp.min(jnp.where(work == m, iota, L))
            onehot = iota == idx
            masks.append(onehot)
            delays.append(idx)
            work = jnp.where(onehot, -jnp.inf, work)

        e = jnp.exp(mv - m0)
        sel = masks[0]
        for mk in masks[1:]:
            sel = jnp.logical_or(sel, mk)
        denom = jnp.sum(jnp.where(sel, e, 0.0))

        agg = jnp.zeros((L, _DM), jnp.float32)
        for i in range(topk):
            w = jnp.sum(jnp.where(masks[i], e, 0.0)) / denom
            agg = agg + w * pltpu.roll(v, (L - delays[i]) % L, 0)

        out_ref[0] = _bmm(agg, wo_ref[...]) + bo_ref[...]

    return body


def _attn(xq, xkv, ap, L):
    topk = int(_FACTOR * math.log(L))
    scale = 1.0 / (L * _DM)
    cf, sf, ci, si = _dft_half_mats(L)
    b2 = lambda v: v.reshape(1, -1)
    args = (xq, xkv, ap['Wq'], b2(ap['bq']), ap['Wk'], b2(ap['bk']),
            ap['Wv'], b2(ap['bv']), ap['Wo'], b2(ap['bo']), cf, sf, ci, si)
    in_specs = [_batch_spec(xq.shape), _batch_spec(xkv.shape)] + \
               [_full_spec(a.shape) for a in args[2:]]
    return pl.pallas_call(
        _make_attn_body(L, topk, scale),
        grid=(xq.shape[0],),
        in_specs=in_specs,
        out_specs=_batch_spec((xq.shape[0], L, _DM)),
        out_shape=jax.ShapeDtypeStruct((xq.shape[0], L, _DM), jnp.float32),
    )(*args)


def _enc_ffn_body(x_ref, a_ref, w1_ref, b1_ref, w2_ref, b2_ref, A_ref,
                  out_ref):
    u = x_ref[0] + a_ref[0]
    x1 = u - _ma_mm(u, A_ref[...])
    h = jax.nn.gelu(_bmm(x1, w1_ref[...]) + b1_ref[...])
    y = _bmm(h, w2_ref[...]) + b2_ref[...]
    s = x1 + y
    out_ref[0] = s - _ma_mm(s, A_ref[...])


def _enc_ffn(x, a, lp):
    Bq, L, D = x.shape
    b2 = lambda v: v.reshape(1, -1)
    args = (x, a, lp['W1'], b2(lp['b1']), lp['W2'], b2(lp['b2']), _ma_mat(L))
    in_specs = [_batch_spec(x.shape), _batch_spec(a.shape)] + \
               [_full_spec(w.shape) for w in args[2:]]
    return pl.pallas_call(
        _enc_ffn_body,
        grid=(Bq,),
        in_specs=in_specs,
        out_specs=_batch_spec((Bq, L, D)),
        out_shape=jax.ShapeDtypeStruct((Bq, L, D), jnp.float32),
    )(*args)


def _add_decomp_body(x_ref, a_ref, A_ref, seas_ref, tr_ref):
    u = x_ref[0] + a_ref[0]
    ma = _ma_mm(u, A_ref[...])
    seas_ref[0] = u - ma
    tr_ref[0] = ma


def _add_decomp(x, a):
    Bq, L, D = x.shape
    A = _ma_mat(L)
    return pl.pallas_call(
        _add_decomp_body,
        grid=(Bq,),
        in_specs=[_batch_spec(x.shape), _batch_spec(a.shape),
                  _full_spec(A.shape)],
        out_specs=[_batch_spec((Bq, L, D)), _batch_spec((Bq, L, D))],
        out_shape=[jax.ShapeDtypeStruct((Bq, L, D), jnp.float32),
                   jax.ShapeDtypeStruct((Bq, L, D), jnp.float32)],
    )(x, a, A)


def _dec_ffn_body(x_ref, w1_ref, b1_ref, w2_ref, b2_ref, A_ref, seas_ref,
                  tr_ref):
    x = x_ref[0]
    h = jax.nn.gelu(_bmm(x, w1_ref[...]) + b1_ref[...])
    y = _bmm(h, w2_ref[...]) + b2_ref[...]
    u = y + x
    ma = _ma_mm(u, A_ref[...])
    seas_ref[0] = u - ma
    tr_ref[0] = ma


def _dec_ffn(x, lp):
    Bq, L, D = x.shape
    b2 = lambda v: v.reshape(1, -1)
    args = (x, lp['W1'], b2(lp['b1']), lp['W2'], b2(lp['b2']), _ma_mat(L))
    in_specs = [_batch_spec(x.shape)] + [_full_spec(w.shape) for w in args[1:]]
    return pl.pallas_call(
        _dec_ffn_body,
        grid=(Bq,),
        in_specs=in_specs,
        out_specs=[_batch_spec((Bq, L, D)), _batch_spec((Bq, L, D))],
        out_shape=[jax.ShapeDtypeStruct((Bq, L, D), jnp.float32),
                   jax.ShapeDtypeStruct((Bq, L, D), jnp.float32)],
    )(*args)


def _trend_body(t1_ref, t2_ref, t3_ref, w3_ref, out_ref):
    t = t1_ref[0] + t2_ref[0] + t3_ref[0]
    tm1 = jnp.concatenate([t[-1:], t[:-1]], axis=0)
    tp1 = jnp.concatenate([t[1:], t[:1]], axis=0)
    out_ref[0] = (_bmm(tm1, w3_ref[0]) + _bmm(t, w3_ref[1])
                  + _bmm(tp1, w3_ref[2]))


def _trend_conv(t1, t2, t3, w3):
    Bq, L, _ = t1.shape
    return pl.pallas_call(
        _trend_body,
        grid=(Bq,),
        in_specs=[_batch_spec(t1.shape), _batch_spec(t2.shape),
                  _batch_spec(t3.shape), _full_spec(w3.shape)],
        out_specs=_batch_spec((Bq, L, _CIN)),
        out_shape=jax.ShapeDtypeStruct((Bq, L, _CIN), jnp.float32),
    )(t1, t2, t3, w3)


def _norm_body(x_ref, g_ref, b_ref, out_ref):
    x = x_ref[0]
    mu = jnp.mean(x, axis=-1, keepdims=True)
    var = jnp.mean((x - mu) ** 2, axis=-1, keepdims=True)
    xh = (x - mu) / jnp.sqrt(var + 1e-5) * g_ref[...] + b_ref[...]
    out_ref[0] = xh - jnp.mean(xh, axis=0, keepdims=True)


def _norm(x, g, b):
    Bq, L, D = x.shape
    g2, b2 = g.reshape(1, -1), b.reshape(1, -1)
    return pl.pallas_call(
        _norm_body,
        grid=(Bq,),
        in_specs=[_batch_spec(x.shape), _full_spec(g2.shape),
                  _full_spec(b2.shape)],
        out_specs=_batch_spec((Bq, L, D)),
        out_shape=jax.ShapeDtypeStruct((Bq, L, D), jnp.float32),
    )(x, g2, b2)


def _final_body(x_ref, g_ref, b_ref, wp_ref, bp_ref, ti_ref, tr_ref, out_ref):
    x = x_ref[0]
    mu = jnp.mean(x, axis=-1, keepdims=True)
    var = jnp.mean((x - mu) ** 2, axis=-1, keepdims=True)
    xh = (x - mu) / jnp.sqrt(var + 1e-5) * g_ref[...] + b_ref[...]
    xh = xh - jnp.mean(xh, axis=0, keepdims=True)
    seasonal = _bmm(xh, wp_ref[...]) + bp_ref[...]
    full = seasonal + ti_ref[0] + tr_ref[0]
    out_ref[0] = full[_LABEL:, :]


def _final(x, g, b, wp, bp, ti, tr):
    Bq, L, D = x.shape
    g2, b2, bp2 = g.reshape(1, -1), b.reshape(1, -1), bp.reshape(1, -1)
    args = (x, g2, b2, wp, bp2, ti, tr)
    in_specs = [_batch_spec(x.shape), _full_spec(g2.shape),
                _full_spec(b2.shape), _full_spec(wp.shape),
                _full_spec(bp2.shape), _batch_spec(ti.shape),
                _batch_spec(tr.shape)]
    return pl.pallas_call(
        _final_body,
        grid=(Bq,),
        in_specs=in_specs,
        out_specs=_batch_spec((Bq, _PRED, _CIN)),
        out_shape=jax.ShapeDtypeStruct((Bq, _PRED, _CIN), jnp.float32),
    )(*args)


def kernel(x_enc, x_mark_enc, x_dec, x_mark_dec, params):
    p = params
    Bq = x_enc.shape[0]

    seas, ma, mean = _head_decomp(x_enc)
    trend_init = jnp.concatenate(
        [ma[:, -_LABEL:, :], jnp.repeat(mean, _PRED, axis=1)], axis=1)
    seasonal_init = jnp.concatenate(
        [seas[:, -_LABEL:, :],
         jnp.zeros((Bq, _PRED, _CIN), jnp.float32)], axis=1)

    enc = _embed(x_enc, x_mark_enc, p['tok_enc'], p['te_enc'],
                 _pos_encoding(_SEQ, _DM))
    for lp in p['enc_layers']:
        a = _attn(enc, enc, lp['attn'], _SEQ)
        enc = _enc_ffn(enc, a, lp)
    enc = _norm(enc, p['enc_norm_g'], p['enc_norm_b'])

    dec = _embed(seasonal_init, x_mark_dec, p['tok_dec'], p['te_dec'],
                 _pos_encoding(_DEC, _DM))
    enc_trunc = enc[:, :_DEC, :]
    trend_total = None
    for lp in p['dec_layers']:
        a1 = _attn(dec, dec, lp['self_attn'], _DEC)
        x1, t1 = _add_decomp(dec, a1)
        a2 = _attn(x1, enc_trunc, lp['cross_attn'], _DEC)
        x2, t2 = _add_decomp(x1, a2)
        dec, t3 = _dec_ffn(x2, lp)
        rt = _trend_conv(t1, t2, t3, lp['Wtrend'])
        trend_total = rt if trend_total is None else trend_total + rt

    return _final(dec, p['dec_norm_g'], p['dec_norm_b'], p['Wproj'],
                  p['bproj'], trend_init, trend_total)


# speedup vs baseline: 15.0133x; 1.0741x over previous
import math

import jax
import jax.numpy as jnp
import numpy as np
from jax.experimental import pallas as pl
from jax.experimental.pallas import tpu as pltpu

_B = 8
_SEQ = 512
_LABEL = 96
_PRED = 192
_DEC = _LABEL + _PRED
_CIN = 321
_DM = 512
_DFF = 2048
_MA = 25
_PAD = (_MA - 1) // 2
_FACTOR = 3

_PREC = jax.lax.Precision.HIGHEST


def _bmm(a, b):
    return jnp.dot(a.astype(jnp.bfloat16), b.astype(jnp.bfloat16),
                   preferred_element_type=jnp.float32)


def _pos_encoding(L, d):
    pe = np.zeros((L, d), dtype=np.float32)
    pos = np.arange(L, dtype=np.float32)[:, None]
    div = np.exp(np.arange(0, d, 2, dtype=np.float32) * (-math.log(10000.0) / d))
    pe[:, 0::2] = np.sin(pos * div)
    pe[:, 1::2] = np.cos(pos * div)
    return jnp.asarray(pe)


def _dft_half_mats(L):
    F = L // 2 + 1
    Fp = ((F + 7) // 8) * 8
    f = np.arange(Fp)
    t = np.arange(L)
    ft = np.outer(f, t) % L
    ang = (2.0 * np.pi / L) * ft.astype(np.float64)
    cf = np.cos(ang)
    sf = np.sin(ang)
    cf[F:] = 0.0
    sf[F:] = 0.0
    alpha = np.full((Fp, 1), 2.0)
    alpha[0] = 1.0
    alpha[F - 1] = 1.0 if L % 2 == 0 else 2.0
    alpha[F:] = 0.0
    ci = (alpha * cf).T
    si = (alpha * sf).T
    return (jnp.asarray(cf, dtype=jnp.float32),
            jnp.asarray(sf, dtype=jnp.float32),
            jnp.asarray(ci, dtype=jnp.float32),
            jnp.asarray(si, dtype=jnp.float32))


def _batch_spec(shape):
    nd = len(shape)
    return pl.BlockSpec((1,) + tuple(shape[1:]),
                        lambda b, _nd=nd: (b,) + (0,) * (_nd - 1))


def _full_spec(shape):
    nd = len(shape)
    return pl.BlockSpec(tuple(shape), lambda b, _nd=nd: (0,) * _nd)


def _ma_mat(L):
    A = np.zeros((L, L), np.float32)
    for t in range(L):
        for j in range(t - _PAD, t + _PAD + 1):
            A[t, min(max(j, 0), L - 1)] += 1.0
    return jnp.asarray(A, jnp.bfloat16)


def _ma_mm(u, A):
    hi = u.astype(jnp.bfloat16)
    r1 = u - hi.astype(jnp.float32)
    mid = r1.astype(jnp.bfloat16)
    lo = (r1 - mid.astype(jnp.float32)).astype(jnp.bfloat16)
    s = (jnp.dot(A, hi, preferred_element_type=jnp.float32)
         + jnp.dot(A, mid, preferred_element_type=jnp.float32)
         + jnp.dot(A, lo, preferred_element_type=jnp.float32))
    return s * (1.0 / _MA)


def _head_body(x_ref, A_ref, seas_ref, ma_ref, mean_ref):
    x = x_ref[0]
    ma = _ma_mm(x, A_ref[...])
    seas_ref[0] = x - ma
    ma_ref[0] = ma
    mean_ref[0] = jnp.mean(x, axis=0, keepdims=True)


def _head_decomp(x):
    Bq, L, C = x.shape
    A = _ma_mat(L)
    return pl.pallas_call(
        _head_body,
        grid=(Bq,),
        in_specs=[_batch_spec(x.shape), _full_spec(A.shape)],
        out_specs=[_batch_spec((Bq, L, C)), _batch_spec((Bq, L, C)),
                   _batch_spec((Bq, 1, C))],
        out_shape=[jax.ShapeDtypeStruct((Bq, L, C), jnp.float32),
                   jax.ShapeDtypeStruct((Bq, L, C), jnp.float32),
                   jax.ShapeDtypeStruct((Bq, 1, C), jnp.float32)],
    )(x, A)


def _embed_body(x_ref, mark_ref, w3_ref, te_ref, pe_ref, out_ref):
    x = x_ref[0]
    xm1 = jnp.concatenate([x[-1:], x[:-1]], axis=0)
    xp1 = jnp.concatenate([x[1:], x[:1]], axis=0)
    acc = _bmm(xm1, w3_ref[0])
    acc = acc + _bmm(x, w3_ref[1])
    acc = acc + _bmm(xp1, w3_ref[2])
    acc = acc + _bmm(mark_ref[0], te_ref[...])
    out_ref[0] = acc + pe_ref[...]


def _embed(x, mark, w3, te, pe):
    Bq, L, _ = x.shape
    return pl.pallas_call(
        _embed_body,
        grid=(Bq,),
        in_specs=[_batch_spec(x.shape), _batch_spec(mark.shape),
                  _full_spec(w3.shape), _full_spec(te.shape),
                  _full_spec(pe.shape)],
        out_specs=_batch_spec((Bq, L, _DM)),
        out_shape=jax.ShapeDtypeStruct((Bq, L, _DM), jnp.float32),
    )(x, mark, w3, te, pe)


def _make_attn_body(L, topk, scale):
    def body(xq_ref, xkv_ref, wq_ref, bq_ref, wk_ref, bk_ref, wv_ref, bv_ref,
             wo_ref, bo_ref, cf_ref, sf_ref, ci_ref, si_ref, out_ref):
        xq = xq_ref[0]
        xkv = xkv_ref[0]
        q = _bmm(xq, wq_ref[...]) + bq_ref[...]
        k = _bmm(xkv, wk_ref[...]) + bk_ref[...]
        v = _bmm(xkv, wv_ref[...]) + bv_ref[...]

        cf = cf_ref[...]
        sf = sf_ref[...]
        qr = jnp.dot(cf, q, precision=_PREC)
        qi = -jnp.dot(sf, q, precision=_PREC)
        kr = jnp.dot(cf, k, precision=_PREC)
        ki = -jnp.dot(sf, k, precision=_PREC)
        pr = jnp.sum(qr * kr + qi * ki, axis=1, keepdims=True)
        pi = jnp.sum(qi * kr - qr * ki, axis=1, keepdims=True)
        mv = (jnp.dot(ci_ref[...], pr, precision=_PREC)
              - jnp.dot(si_ref[...], pi, precision=_PREC)) * scale

        iota = jax.lax.broadcasted_iota(jnp.int32, (L, 1), 0)
        work = mv
        sel = jnp.zeros((L, 1), jnp.bool_)
        m0 = None
        for i in range(topk):
            m = jnp.max(work)
            if i == 0:
                m0 = m
            idx = jnp.min(jnp.where(work == m, iota, L))
            onehot = iota == idx
            sel = jnp.logical_or(sel, onehot)
            work = jnp.where(onehot, -jnp.inf, work)

        e = jnp.where(sel, jnp.exp(mv - m0), 0.0)
        cvec = e / jnp.sum(e)

        cr = jnp.dot(cf, cvec, precision=_PREC)
        cm = -jnp.dot(sf, cvec, precision=_PREC)
        vr = jnp.dot(cf, v, precision=_PREC)
        vi = -jnp.dot(sf, v, precision=_PREC)
        re = cr * vr + cm * vi
        im = cr * vi - cm * vr
        agg = (jnp.dot(ci_ref[...], re, precision=_PREC)
               - jnp.dot(si_ref[...], im, precision=_PREC)) * (1.0 / L)

        out_ref[0] = _bmm(agg, wo_ref[...]) + bo_ref[...]

    return body


def _attn(xq, xkv, ap, L):
    topk = int(_FACTOR * math.log(L))
    scale = 1.0 / (L * _DM)
    cf, sf, ci, si = _dft_half_mats(L)
    b2 = lambda v: v.reshape(1, -1)
    args = (xq, xkv, ap['Wq'], b2(ap['bq']), ap['Wk'], b2(ap['bk']),
            ap['Wv'], b2(ap['bv']), ap['Wo'], b2(ap['bo']), cf, sf, ci, si)
    in_specs = [_batch_spec(xq.shape), _batch_spec(xkv.shape)] + \
               [_full_spec(a.shape) for a in args[2:]]
    return pl.pallas_call(
        _make_attn_body(L, topk, scale),
        grid=(xq.shape[0],),
        in_specs=in_specs,
        out_specs=_batch_spec((xq.shape[0], L, _DM)),
        out_shape=jax.ShapeDtypeStruct((xq.shape[0], L, _DM), jnp.float32),
    )(*args)


def _enc_ffn_body(x_ref, a_ref, w1_ref, b1_ref, w2_ref, b2_ref, A_ref,
                  out_ref):
    u = x_ref[0] + a_ref[0]
    x1 = u - _ma_mm(u, A_ref[...])
    h = jax.nn.gelu(_bmm(x1, w1_ref[...]) + b1_ref[...])
    y = _bmm(h, w2_ref[...]) + b2_ref[...]
    s = x1 + y
    out_ref[0] = s - _ma_mm(s, A_ref[...])


def _enc_ffn(x, a, lp):
    Bq, L, D = x.shape
    b2 = lambda v: v.reshape(1, -1)
    args = (x, a, lp['W1'], b2(lp['b1']), lp['W2'], b2(lp['b2']), _ma_mat(L))
    in_specs = [_batch_spec(x.shape), _batch_spec(a.shape)] + \
               [_full_spec(w.shape) for w in args[2:]]
    return pl.pallas_call(
        _enc_ffn_body,
        grid=(Bq,),
        in_specs=in_specs,
        out_specs=_batch_spec((Bq, L, D)),
        out_shape=jax.ShapeDtypeStruct((Bq, L, D), jnp.float32),
    )(*args)


def _add_decomp_body(x_ref, a_ref, A_ref, seas_ref, tr_ref):
    u = x_ref[0] + a_ref[0]
    ma = _ma_mm(u, A_ref[...])
    seas_ref[0] = u - ma
    tr_ref[0] = ma


def _add_decomp(x, a):
    Bq, L, D = x.shape
    A = _ma_mat(L)
    return pl.pallas_call(
        _add_decomp_body,
        grid=(Bq,),
        in_specs=[_batch_spec(x.shape), _batch_spec(a.shape),
                  _full_spec(A.shape)],
        out_specs=[_batch_spec((Bq, L, D)), _batch_spec((Bq, L, D))],
        out_shape=[jax.ShapeDtypeStruct((Bq, L, D), jnp.float32),
                   jax.ShapeDtypeStruct((Bq, L, D), jnp.float32)],
    )(x, a, A)


def _dec_ffn_body(x_ref, w1_ref, b1_ref, w2_ref, b2_ref, A_ref, seas_ref,
                  tr_ref):
    x = x_ref[0]
    h = jax.nn.gelu(_bmm(x, w1_ref[...]) + b1_ref[...])
    y = _bmm(h, w2_ref[...]) + b2_ref[...]
    u = y + x
    ma = _ma_mm(u, A_ref[...])
    seas_ref[0] = u - ma
    tr_ref[0] = ma


def _dec_ffn(x, lp):
    Bq, L, D = x.shape
    b2 = lambda v: v.reshape(1, -1)
    args = (x, lp['W1'], b2(lp['b1']), lp['W2'], b2(lp['b2']), _ma_mat(L))
    in_specs = [_batch_spec(x.shape)] + [_full_spec(w.shape) for w in args[1:]]
    return pl.pallas_call(
        _dec_ffn_body,
        grid=(Bq,),
        in_specs=in_specs,
        out_specs=[_batch_spec((Bq, L, D)), _batch_spec((Bq, L, D))],
        out_shape=[jax.ShapeDtypeStruct((Bq, L, D), jnp.float32),
                   jax.ShapeDtypeStruct((Bq, L, D), jnp.float32)],
    )(*args)


def _trend_body(t1_ref, t2_ref, t3_ref, w3_ref, out_ref):
    t = t1_ref[0] + t2_ref[0] + t3_ref[0]
    tm1 = jnp.concatenate([t[-1:], t[:-1]], axis=0)
    tp1 = jnp.concatenate([t[1:], t[:1]], axis=0)
    out_ref[0] = (_bmm(tm1, w3_ref[0]) + _bmm(t, w3_ref[1])
                  + _bmm(tp1, w3_ref[2]))


def _trend_conv(t1, t2, t3, w3):
    Bq, L, _ = t1.shape
    return pl.pallas_call(
        _trend_body,
        grid=(Bq,),
        in_specs=[_batch_spec(t1.shape), _batch_spec(t2.shape),
                  _batch_spec(t3.shape), _full_spec(w3.shape)],
        out_specs=_batch_spec((Bq, L, _CIN)),
        out_shape=jax.ShapeDtypeStruct((Bq, L, _CIN), jnp.float32),
    )(t1, t2, t3, w3)


def _norm_body(x_ref, g_ref, b_ref, out_ref):
    x = x_ref[0]
    mu = jnp.mean(x, axis=-1, keepdims=True)
    var = jnp.mean((x - mu) ** 2, axis=-1, keepdims=True)
    xh = (x - mu) / jnp.sqrt(var + 1e-5) * g_ref[...] + b_ref[...]
    out_ref[0] = xh - jnp.mean(xh, axis=0, keepdims=True)


def _norm(x, g, b):
    Bq, L, D = x.shape
    g2, b2 = g.reshape(1, -1), b.reshape(1, -1)
    return pl.pallas_call(
        _norm_body,
        grid=(Bq,),
        in_specs=[_batch_spec(x.shape), _full_spec(g2.shape),
                  _full_spec(b2.shape)],
        out_specs=_batch_spec((Bq, L, D)),
        out_shape=jax.ShapeDtypeStruct((Bq, L, D), jnp.float32),
    )(x, g2, b2)


def _final_body(x_ref, g_ref, b_ref, wp_ref, bp_ref, ti_ref, tr_ref, out_ref):
    x = x_ref[0]
    mu = jnp.mean(x, axis=-1, keepdims=True)
    var = jnp.mean((x - mu) ** 2, axis=-1, keepdims=True)
    xh = (x - mu) / jnp.sqrt(var + 1e-5) * g_ref[...] + b_ref[...]
    xh = xh - jnp.mean(xh, axis=0, keepdims=True)
    seasonal = _bmm(xh, wp_ref[...]) + bp_ref[...]
    full = seasonal + ti_ref[0] + tr_ref[0]
    out_ref[0] = full[_LABEL:, :]


def _final(x, g, b, wp, bp, ti, tr):
    Bq, L, D = x.shape
    g2, b2, bp2 = g.reshape(1, -1), b.reshape(1, -1), bp.reshape(1, -1)
    args = (x, g2, b2, wp, bp2, ti, tr)
    in_specs = [_batch_spec(x.shape), _full_spec(g2.shape),
                _full_spec(b2.shape), _full_spec(wp.shape),
                _full_spec(bp2.shape), _batch_spec(ti.shape),
                _batch_spec(tr.shape)]
    return pl.pallas_call(
        _final_body,
        grid=(Bq,),
        in_specs=in_specs,
        out_specs=_batch_spec((Bq, _PRED, _CIN)),
        out_shape=jax.ShapeDtypeStruct((Bq, _PRED, _CIN), jnp.float32),
    )(*args)


def kernel(x_enc, x_mark_enc, x_dec, x_mark_dec, params):
    p = params
    Bq = x_enc.shape[0]

    seas, ma, mean = _head_decomp(x_enc)
    trend_init = jnp.concatenate(
        [ma[:, -_LABEL:, :], jnp.repeat(mean, _PRED, axis=1)], axis=1)
    seasonal_init = jnp.concatenate(
        [seas[:, -_LABEL:, :],
         jnp.zeros((Bq, _PRED, _CIN), jnp.float32)], axis=1)

    enc = _embed(x_enc, x_mark_enc, p['tok_enc'], p['te_enc'],
                 _pos_encoding(_SEQ, _DM))
    for lp in p['enc_layers']:
        a = _attn(enc, enc, lp['attn'], _SEQ)
        enc = _enc_ffn(enc, a, lp)
    enc = _norm(enc, p['enc_norm_g'], p['enc_norm_b'])

    dec = _embed(seasonal_init, x_mark_dec, p['tok_dec'], p['te_dec'],
                 _pos_encoding(_DEC, _DM))
    enc_trunc = enc[:, :_DEC, :]
    trend_total = None
    for lp in p['dec_layers']:
        a1 = _attn(dec, dec, lp['self_attn'], _DEC)
        x1, t1 = _add_decomp(dec, a1)
        a2 = _attn(x1, enc_trunc, lp['cross_attn'], _DEC)
        x2, t2 = _add_decomp(x1, a2)
        dec, t3 = _dec_ffn(x2, lp)
        rt = _trend_conv(t1, t2, t3, lp['Wtrend'])
        trend_total = rt if trend_total is None else trend_total + rt

    return _final(dec, p['dec_norm_g'], p['dec_norm_b'], p['Wproj'],
                  p['bproj'], trend_init, trend_total)
